# baseline (speedup 1.0000x reference)
import functools
import jax
import jax.numpy as jnp
from jax import lax
from jax.experimental import pallas as pl
from jax.experimental.pallas import tpu as pltpu
from jax.experimental.pallas import tpu_sc as plsc

N = 10000
E = 320000
HID = 16
PAD = 1024

NC = 2
NS = 16

_HALF16 = [10368] * 3 + [9936] * 12 + [9664]
_HALF128 = [10080] * 11 + [9840] * 4 + [9760]


def _bounds(half):
    b = [0]
    for v in half + half:
        b.append(b[-1] + v)
    assert b[-1] == E
    return b + [0] * (40 - len(b))


def _make_agg_body(d, ch, lrows):
    dsub = d // 16
    zrpt = N // NS

    def body(h_hbm, src_hbm, dst_hbm, bnd_hbm, out_hbm, bnd_v, sidx_v, didx_v,
             msgs_v, acc_v, prev_v, swr_v, ids_all_v, row_wr_v, rows_all_v,
             stash_id_sh, stash_row_sh, gsem):
        c = lax.axis_index("c")
        s = lax.axis_index("s")
        w = c * NS + s

        zrow = jnp.zeros((16,), jnp.float32)

        def zacc(r, _):
            for k in range(dsub):
                acc_v[r, pl.ds(16 * k, 16)] = zrow
            return 0

        lax.fori_loop(0, lrows, zacc, 0)

        done = 0
        while done < zrpt:
            step = min(lrows, zrpt - done)
            pltpu.sync_copy(acc_v.at[pl.ds(0, step)],
                            out_hbm.at[c, pl.ds(s * zrpt + done, step)])
            done += step

        pltpu.sync_copy(bnd_hbm, bnd_v.at[pl.ds(0, 40)])
        bv = bnd_v[pl.ds(w, 16)]
        lo = pl.multiple_of(bv[0], 8)
        hi = bv[1]
        n_upd = hi - lo

        pltpu.sync_copy(dst_hbm.at[pl.ds(lo, 8)], prev_v.at[pl.ds(8, 8)])

        @pl.when(s > 0)
        def _():
            pltpu.sync_copy(dst_hbm.at[pl.ds(pl.multiple_of(lo - 8, 8), 8)],
                            prev_v.at[pl.ds(0, 8)])

        pv = prev_v[...]
        first_row = pv[8]
        shared = jnp.logical_and(s > 0, pv[7] == first_row)
        base_row = first_row

        iota = lax.iota(jnp.int32, 16)

        def chunk(i, _):
            off = pl.multiple_of(lo + i * ch, 8)
            pltpu.sync_copy(src_hbm.at[pl.ds(off, ch)], sidx_v)
            pltpu.sync_copy(dst_hbm.at[pl.ds(off, ch)], didx_v.at[pl.ds(0, ch)])
            pltpu.async_copy(h_hbm.at[sidx_v], msgs_v, gsem).wait()
            nin = jnp.minimum(n_upd - i * ch, ch)

            def upd(e, _):
                dv = didx_v[pl.ds(e, 16)]
                rowv = jnp.zeros((16,), jnp.int32) + (dv[0] - base_row)
                for k in range(dsub):
                    msg = msgs_v[e, pl.ds(16 * k, 16)]
                    plsc.addupdate_scatter(acc_v, [rowv, iota + (16 * k)], msg)
                return 0

            lax.fori_loop(0, nin, upd, 0)
            return 0

        nchunks = (n_upd + ch - 1) // ch
        lax.fori_loop(0, nchunks, chunk, 0)

        lv = didx_v[pl.ds((n_upd - 1) % ch, 16)]
        last_row = lv[0]

        sid = jnp.where(shared, first_row, -1)
        swr_v[pl.ds(0, 16)] = jnp.zeros((16,), jnp.int32) + sid
        pltpu.sync_copy(swr_v, stash_id_sh.at[s])
        for k in range(dsub):
            row_wr_v[pl.ds(16 * k, 16)] = acc_v[0, pl.ds(16 * k, 16)]
        pltpu.sync_copy(row_wr_v, stash_row_sh.at[s])
        plsc.subcore_barrier()

        start = jnp.where(shared, 1, 0)
        nrows = last_row - base_row + 1 - start

        def wr8(i, _):
            r = start + i * 8
            pltpu.sync_copy(acc_v.at[pl.ds(r, 8)],
                            out_hbm.at[c, pl.ds(base_row + r, 8)])
            return 0

        lax.fori_loop(0, nrows // 8, wr8, 0)

        def wr1(i, _):
            r = start + (nrows // 8) * 8 + i
            pltpu.sync_copy(acc_v.at[pl.ds(r, 1)],
                            out_hbm.at[c, pl.ds(base_row + r, 1)])
            return 0

        lax.fori_loop(0, nrows % 8, wr1, 0)

        pltpu.sync_copy(stash_id_sh, ids_all_v)
        pltpu.sync_copy(stash_row_sh, rows_all_v)
        lr_last = last_row - base_row
        zvec = jnp.zeros((16,), jnp.float32)
        merged = [acc_v[lr_last, pl.ds(16 * k, 16)] for k in range(dsub)]
        for u in range(1, NS):
            su = jnp.minimum(s + u, NS - 1)
            idrow = ids_all_v[su]
            cond = jnp.logical_and(s + u < NS, idrow[0] == last_row)
            for k in range(dsub):
                contrib = jnp.where(cond, rows_all_v[su, pl.ds(16 * k, 16)],
                                    zvec)
                merged[k] = merged[k] + contrib

        degenerate = jnp.logical_and(shared, first_row == last_row)

        @pl.when(jnp.logical_not(degenerate))
        def _():
            for k in range(dsub):
                row_wr_v[pl.ds(16 * k, 16)] = merged[k]
            pltpu.sync_copy(row_wr_v, out_hbm.at[c, last_row])

    return body


def _make_sc_aggregate(d, ch, lrows, half):
    body = _make_agg_body(d, ch, lrows)
    bnd = jnp.asarray(_bounds(half), jnp.int32)

    @jax.jit
    def agg(h, ssrc, sdst):
        mesh = plsc.VectorSubcoreMesh(core_axis_name="c", subcore_axis_name="s")
        return pl.kernel(
            body,
            out_type=jax.ShapeDtypeStruct((NC, N, d), jnp.float32),
            mesh=mesh,
            compiler_params=pltpu.CompilerParams(use_tc_tiling_on_sc=False, needs_layout_passes=False),
            scratch_types=[
                pltpu.VMEM((48,), jnp.int32),
                pltpu.VMEM((ch,), jnp.int32),
                pltpu.VMEM((ch + 16,), jnp.int32),
                pltpu.VMEM((ch, d), jnp.float32),
                pltpu.VMEM((lrows, d), jnp.float32),
                pltpu.VMEM((16,), jnp.int32),
                pltpu.VMEM((16,), jnp.int32),
                pltpu.VMEM((NS, 16), jnp.int32),
                pltpu.VMEM((d,), jnp.float32),
                pltpu.VMEM((NS, d), jnp.float32),
                pltpu.VMEM_SHARED((NS, 16), jnp.int32),
                pltpu.VMEM_SHARED((NS, d), jnp.float32),
                pltpu.SemaphoreType.DMA,
            ],
        )(h, ssrc, sdst, bnd)

    return agg


_sc_aggregate16 = _make_sc_aggregate(HID, 1000, 2048, _HALF16)
_sc_aggregate128 = _make_sc_aggregate(128, 80, 512, _HALF128)


BR = 2000


def _mlp0_body(x_ref, agg_ref, w1_ref, b1_ref, w2_ref, b2_ref, o_ref):
    z = x_ref[...] + (agg_ref[0] + agg_ref[1])
    t = jnp.maximum(jnp.dot(z, w1_ref[...], preferred_element_type=jnp.float32)
                    + b1_ref[...], 0.0)
    h = jnp.dot(t, w2_ref[...], preferred_element_type=jnp.float32) + b2_ref[...]
    o_ref[...] = jnp.maximum(h, 0.0)


@jax.jit
def _tc_mlp0(x, agg, w1, b1, w2, b2):
    d = x.shape[1]
    return pl.pallas_call(
        _mlp0_body,
        grid=(N // BR,),
        in_specs=[
            pl.BlockSpec((BR, d), lambda i: (i, 0)),
            pl.BlockSpec((NC, BR, d), lambda i: (0, i, 0)),
            pl.BlockSpec((d, HID), lambda i: (0, 0)),
            pl.BlockSpec((1, HID), lambda i: (0, 0)),
            pl.BlockSpec((HID, HID), lambda i: (0, 0)),
            pl.BlockSpec((1, HID), lambda i: (0, 0)),
        ],
        out_specs=pl.BlockSpec((BR, HID), lambda i: (i, 0)),
        out_shape=jax.ShapeDtypeStruct((N, HID), jnp.float32),
    )(x, agg, w1, b1, w2, b2)


def _mlp_body(h_ref, agg_ref, w1_ref, b1_ref, w2_ref, b2_ref, o_ref, *, relu):
    z = h_ref[...] + (agg_ref[0] + agg_ref[1])
    t = jnp.maximum(jnp.dot(z, w1_ref[...], preferred_element_type=jnp.float32)
                    + b1_ref[...], 0.0)
    h = jnp.dot(t, w2_ref[...], preferred_element_type=jnp.float32) + b2_ref[...]
    if relu:
        h = jnp.maximum(h, 0.0)
    o_ref[...] = h


@functools.partial(jax.jit, static_argnames=("relu",))
def _tc_mlp(h, agg, w1, b1, w2, b2, relu):
    return pl.pallas_call(
        functools.partial(_mlp_body, relu=relu),
        grid=(N // BR,),
        in_specs=[
            pl.BlockSpec((BR, HID), lambda i: (i, 0)),
            pl.BlockSpec((NC, BR, HID), lambda i: (0, i, 0)),
            pl.BlockSpec((HID, HID), lambda i: (0, 0)),
            pl.BlockSpec((1, HID), lambda i: (0, 0)),
            pl.BlockSpec((HID, HID), lambda i: (0, 0)),
            pl.BlockSpec((1, HID), lambda i: (0, 0)),
        ],
        out_specs=pl.BlockSpec((BR, HID), lambda i: (i, 0)),
        out_shape=jax.ShapeDtypeStruct((N, HID), jnp.float32),
    )(h, agg, w1, b1, w2, b2)


def _mlp_head_body(h_ref, agg_ref, w1_ref, b1_ref, w2_ref, b2_ref,
                   wm_ref, bm_ref, ws_ref, bs_ref, mean_ref, std_ref):
    z = h_ref[...] + (agg_ref[0] + agg_ref[1])
    t = jnp.maximum(jnp.dot(z, w1_ref[...], preferred_element_type=jnp.float32)
                    + b1_ref[...], 0.0)
    h = jnp.dot(t, w2_ref[...], preferred_element_type=jnp.float32) + b2_ref[...]
    mean_ref[...] = jnp.dot(h, wm_ref[...],
                            preferred_element_type=jnp.float32) + bm_ref[...]
    y = jnp.dot(h, ws_ref[...], preferred_element_type=jnp.float32) + bs_ref[...]
    std_ref[...] = jnp.maximum(y, 0.0) + jnp.log1p(jnp.exp(-jnp.abs(y)))


@jax.jit
def _tc_mlp_head(h, agg, w1, b1, w2, b2, wm, bm, ws, bs):
    lat = wm.shape[1]
    return pl.pallas_call(
        _mlp_head_body,
        grid=(N // BR,),
        in_specs=[
            pl.BlockSpec((BR, HID), lambda i: (i, 0)),
            pl.BlockSpec((NC, BR, HID), lambda i: (0, i, 0)),
            pl.BlockSpec((HID, HID), lambda i: (0, 0)),
            pl.BlockSpec((1, HID), lambda i: (0, 0)),
            pl.BlockSpec((HID, HID), lambda i: (0, 0)),
            pl.BlockSpec((1, HID), lambda i: (0, 0)),
            pl.BlockSpec((HID, lat), lambda i: (0, 0)),
            pl.BlockSpec((1, lat), lambda i: (0, 0)),
            pl.BlockSpec((HID, lat), lambda i: (0, 0)),
            pl.BlockSpec((1, lat), lambda i: (0, 0)),
        ],
        out_specs=[
            pl.BlockSpec((BR, lat), lambda i: (i, 0)),
            pl.BlockSpec((BR, lat), lambda i: (i, 0)),
        ],
        out_shape=[
            jax.ShapeDtypeStruct((N, lat), jnp.float32),
            jax.ShapeDtypeStruct((N, lat), jnp.float32),
        ],
    )(h, agg, w1, b1, w2, b2, wm, bm, ws, bs)


def kernel(x, edge_index, W1_0, b1_0, W2_0, b2_0, W1s, b1s, W2s, b2s, Wm, bm, Ws, bs):
    src = jnp.asarray(edge_index[0], jnp.int32)
    dst = jnp.asarray(edge_index[1], jnp.int32)
    perm = jnp.argsort(dst, stable=True)
    zpad = jnp.zeros((PAD,), jnp.int32)
    ssrc = jnp.concatenate([src[perm], zpad])
    sdst = jnp.concatenate([dst[perm], zpad])

    agg = _sc_aggregate128(x, ssrc, sdst)
    h = _tc_mlp0(x, agg, W1_0, b1_0.reshape(1, HID), W2_0, b2_0.reshape(1, HID))

    n_rest = W1s.shape[0]
    for i in range(n_rest - 1):
        agg = _sc_aggregate16(h, ssrc, sdst)
        h = _tc_mlp(h, agg, W1s[i], b1s[i].reshape(1, HID), W2s[i],
                    b2s[i].reshape(1, HID), relu=True)

    i = n_rest - 1
    agg = _sc_aggregate16(h, ssrc, sdst)
    lat = Wm.shape[1]
    mean, std = _tc_mlp_head(h, agg, W1s[i], b1s[i].reshape(1, HID), W2s[i],
                             b2s[i].reshape(1, HID), Wm, bm.reshape(1, lat),
                             Ws, bs.reshape(1, lat))
    return (mean, std)


# speedup vs baseline: 1.6767x; 1.6767x over previous
import functools
import jax
import jax.numpy as jnp
from jax import lax
from jax.experimental import pallas as pl
from jax.experimental.pallas import tpu as pltpu
from jax.experimental.pallas import tpu_sc as plsc

N = 10000
E = 320000
HID = 16
PAD = 1024

NC = 2
NS = 16

_HALF16 = [10368] * 3 + [9936] * 12 + [9664]
_HALF128 = [10080] * 11 + [9840] * 4 + [9760]


def _bounds(half):
    b = [0]
    for v in half + half:
        b.append(b[-1] + v)
    assert b[-1] == E
    return b + [0] * (40 - len(b))


def _make_agg_body(d, ch, lrows):
    dsub = d // 16
    zrpt = N // NS

    def body(h_hbm, src_hbm, dst_hbm, bnd_hbm, out_hbm, bnd_v, sidx_v, didx_v,
             msgs_v, acc_v, prev_v, swr_v, ids_all_v, row_wr_v, rows_all_v,
             stash_id_sh, stash_row_sh, gsem):
        c = lax.axis_index("c")
        s = lax.axis_index("s")
        w = c * NS + s

        zrow = jnp.zeros((16,), jnp.float32)

        def zacc(r, _):
            for k in range(dsub):
                acc_v[r, pl.ds(16 * k, 16)] = zrow
            return 0

        lax.fori_loop(0, lrows, zacc, 0)

        done = 0
        while done < zrpt:
            step = min(lrows, zrpt - done)
            pltpu.sync_copy(acc_v.at[pl.ds(0, step)],
                            out_hbm.at[c, pl.ds(s * zrpt + done, step)])
            done += step

        pltpu.sync_copy(bnd_hbm, bnd_v.at[pl.ds(0, 40)])
        bv = bnd_v[pl.ds(w, 16)]
        lo = pl.multiple_of(bv[0], 8)
        hi = bv[1]
        n_upd = hi - lo

        pltpu.sync_copy(dst_hbm.at[pl.ds(lo, 8)], prev_v.at[pl.ds(8, 8)])

        @pl.when(s > 0)
        def _():
            pltpu.sync_copy(dst_hbm.at[pl.ds(pl.multiple_of(lo - 8, 8), 8)],
                            prev_v.at[pl.ds(0, 8)])

        pv = prev_v[...]
        first_row = pv[8]
        shared = jnp.logical_and(s > 0, pv[7] == first_row)
        base_row = first_row

        zvec16 = jnp.zeros((16,), jnp.float32)

        def flush(row, accs):
            lr = row - base_row
            for k in range(dsub):
                acc_v[lr, pl.ds(16 * k, 16)] = accs[k]

        def chunk(i, carry):
            off = pl.multiple_of(lo + i * ch, 8)
            pltpu.sync_copy(src_hbm.at[pl.ds(off, ch)], sidx_v)
            pltpu.sync_copy(dst_hbm.at[pl.ds(off, ch)], didx_v.at[pl.ds(0, ch)])
            pltpu.async_copy(h_hbm.at[sidx_v], msgs_v, gsem).wait()
            nin = jnp.minimum(n_upd - i * ch, ch)

            def block(b, carry):
                cur_row = carry[0]
                accs = list(carry[1:])
                dv = didx_v[pl.ds(b * 16, 16)]
                for j in range(16):
                    nd = dv[j]
                    change = nd != cur_row

                    @pl.when(change)
                    def _(cur_row=cur_row, accs=tuple(accs)):
                        flush(cur_row, accs)

                    for k in range(dsub):
                        msg = msgs_v[b * 16 + j, pl.ds(16 * k, 16)]
                        accs[k] = jnp.where(change, msg, accs[k] + msg)
                    cur_row = nd
                return (cur_row, *accs)

            return lax.fori_loop(0, nin // 16, block, carry)

        nchunks = (n_upd + ch - 1) // ch
        carry0 = (first_row, *([zvec16] * dsub))
        carry = lax.fori_loop(0, nchunks, chunk, carry0)
        last_row = carry[0]
        flush(last_row, carry[1:])

        sid = jnp.where(shared, first_row, -1)
        swr_v[pl.ds(0, 16)] = jnp.zeros((16,), jnp.int32) + sid
        pltpu.sync_copy(swr_v, stash_id_sh.at[s])
        for k in range(dsub):
            row_wr_v[pl.ds(16 * k, 16)] = acc_v[0, pl.ds(16 * k, 16)]
        pltpu.sync_copy(row_wr_v, stash_row_sh.at[s])
        plsc.subcore_barrier()

        start = jnp.where(shared, 1, 0)
        nrows = last_row - base_row + 1 - start

        def wr8(i, _):
            r = start + i * 8
            pltpu.sync_copy(acc_v.at[pl.ds(r, 8)],
                            out_hbm.at[c, pl.ds(base_row + r, 8)])
            return 0

        lax.fori_loop(0, nrows // 8, wr8, 0)

        def wr1(i, _):
            r = start + (nrows // 8) * 8 + i
            pltpu.sync_copy(acc_v.at[pl.ds(r, 1)],
                            out_hbm.at[c, pl.ds(base_row + r, 1)])
            return 0

        lax.fori_loop(0, nrows % 8, wr1, 0)

        pltpu.sync_copy(stash_id_sh, ids_all_v)
        pltpu.sync_copy(stash_row_sh, rows_all_v)
        lr_last = last_row - base_row
        zvec = jnp.zeros((16,), jnp.float32)
        merged = [acc_v[lr_last, pl.ds(16 * k, 16)] for k in range(dsub)]
        for u in range(1, NS):
            su = jnp.minimum(s + u, NS - 1)
            idrow = ids_all_v[su]
            cond = jnp.logical_and(s + u < NS, idrow[0] == last_row)
            for k in range(dsub):
                contrib = jnp.where(cond, rows_all_v[su, pl.ds(16 * k, 16)],
                                    zvec)
                merged[k] = merged[k] + contrib

        degenerate = jnp.logical_and(shared, first_row == last_row)

        @pl.when(jnp.logical_not(degenerate))
        def _():
            for k in range(dsub):
                row_wr_v[pl.ds(16 * k, 16)] = merged[k]
            pltpu.sync_copy(row_wr_v, out_hbm.at[c, last_row])

    return body


def _make_sc_aggregate(d, ch, lrows, half):
    body = _make_agg_body(d, ch, lrows)
    bnd = jnp.asarray(_bounds(half), jnp.int32)

    @jax.jit
    def agg(h, ssrc, sdst):
        mesh = plsc.VectorSubcoreMesh(core_axis_name="c", subcore_axis_name="s")
        return pl.kernel(
            body,
            out_type=jax.ShapeDtypeStruct((NC, N, d), jnp.float32),
            mesh=mesh,
            compiler_params=pltpu.CompilerParams(use_tc_tiling_on_sc=False, needs_layout_passes=False),
            scratch_types=[
                pltpu.VMEM((48,), jnp.int32),
                pltpu.VMEM((ch,), jnp.int32),
                pltpu.VMEM((ch + 16,), jnp.int32),
                pltpu.VMEM((ch, d), jnp.float32),
                pltpu.VMEM((lrows, d), jnp.float32),
                pltpu.VMEM((16,), jnp.int32),
                pltpu.VMEM((16,), jnp.int32),
                pltpu.VMEM((NS, 16), jnp.int32),
                pltpu.VMEM((d,), jnp.float32),
                pltpu.VMEM((NS, d), jnp.float32),
                pltpu.VMEM_SHARED((NS, 16), jnp.int32),
                pltpu.VMEM_SHARED((NS, d), jnp.float32),
                pltpu.SemaphoreType.DMA,
            ],
        )(h, ssrc, sdst, bnd)

    return agg


_sc_aggregate16 = _make_sc_aggregate(HID, 960, 2048, _HALF16)
_sc_aggregate128 = _make_sc_aggregate(128, 80, 512, _HALF128)


BR = 2000


def _mlp0_body(x_ref, agg_ref, w1_ref, b1_ref, w2_ref, b2_ref, o_ref):
    z = x_ref[...] + (agg_ref[0] + agg_ref[1])
    t = jnp.maximum(jnp.dot(z, w1_ref[...], preferred_element_type=jnp.float32)
                    + b1_ref[...], 0.0)
    h = jnp.dot(t, w2_ref[...], preferred_element_type=jnp.float32) + b2_ref[...]
    o_ref[...] = jnp.maximum(h, 0.0)


@jax.jit
def _tc_mlp0(x, agg, w1, b1, w2, b2):
    d = x.shape[1]
    return pl.pallas_call(
        _mlp0_body,
        grid=(N // BR,),
        in_specs=[
            pl.BlockSpec((BR, d), lambda i: (i, 0)),
            pl.BlockSpec((NC, BR, d), lambda i: (0, i, 0)),
            pl.BlockSpec((d, HID), lambda i: (0, 0)),
            pl.BlockSpec((1, HID), lambda i: (0, 0)),
            pl.BlockSpec((HID, HID), lambda i: (0, 0)),
            pl.BlockSpec((1, HID), lambda i: (0, 0)),
        ],
        out_specs=pl.BlockSpec((BR, HID), lambda i: (i, 0)),
        out_shape=jax.ShapeDtypeStruct((N, HID), jnp.float32),
    )(x, agg, w1, b1, w2, b2)


def _mlp_body(h_ref, agg_ref, w1_ref, b1_ref, w2_ref, b2_ref, o_ref, *, relu):
    z = h_ref[...] + (agg_ref[0] + agg_ref[1])
    t = jnp.maximum(jnp.dot(z, w1_ref[...], preferred_element_type=jnp.float32)
                    + b1_ref[...], 0.0)
    h = jnp.dot(t, w2_ref[...], preferred_element_type=jnp.float32) + b2_ref[...]
    if relu:
        h = jnp.maximum(h, 0.0)
    o_ref[...] = h


@functools.partial(jax.jit, static_argnames=("relu",))
def _tc_mlp(h, agg, w1, b1, w2, b2, relu):
    return pl.pallas_call(
        functools.partial(_mlp_body, relu=relu),
        grid=(N // BR,),
        in_specs=[
            pl.BlockSpec((BR, HID), lambda i: (i, 0)),
            pl.BlockSpec((NC, BR, HID), lambda i: (0, i, 0)),
            pl.BlockSpec((HID, HID), lambda i: (0, 0)),
            pl.BlockSpec((1, HID), lambda i: (0, 0)),
            pl.BlockSpec((HID, HID), lambda i: (0, 0)),
            pl.BlockSpec((1, HID), lambda i: (0, 0)),
        ],
        out_specs=pl.BlockSpec((BR, HID), lambda i: (i, 0)),
        out_shape=jax.ShapeDtypeStruct((N, HID), jnp.float32),
    )(h, agg, w1, b1, w2, b2)


def _mlp_head_body(h_ref, agg_ref, w1_ref, b1_ref, w2_ref, b2_ref,
                   wm_ref, bm_ref, ws_ref, bs_ref, mean_ref, std_ref):
    z = h_ref[...] + (agg_ref[0] + agg_ref[1])
    t = jnp.maximum(jnp.dot(z, w1_ref[...], preferred_element_type=jnp.float32)
                    + b1_ref[...], 0.0)
    h = jnp.dot(t, w2_ref[...], preferred_element_type=jnp.float32) + b2_ref[...]
    mean_ref[...] = jnp.dot(h, wm_ref[...],
                            preferred_element_type=jnp.float32) + bm_ref[...]
    y = jnp.dot(h, ws_ref[...], preferred_element_type=jnp.float32) + bs_ref[...]
    std_ref[...] = jnp.maximum(y, 0.0) + jnp.log1p(jnp.exp(-jnp.abs(y)))


@jax.jit
def _tc_mlp_head(h, agg, w1, b1, w2, b2, wm, bm, ws, bs):
    lat = wm.shape[1]
    return pl.pallas_call(
        _mlp_head_body,
        grid=(N // BR,),
        in_specs=[
            pl.BlockSpec((BR, HID), lambda i: (i, 0)),
            pl.BlockSpec((NC, BR, HID), lambda i: (0, i, 0)),
            pl.BlockSpec((HID, HID), lambda i: (0, 0)),
            pl.BlockSpec((1, HID), lambda i: (0, 0)),
            pl.BlockSpec((HID, HID), lambda i: (0, 0)),
            pl.BlockSpec((1, HID), lambda i: (0, 0)),
            pl.BlockSpec((HID, lat), lambda i: (0, 0)),
            pl.BlockSpec((1, lat), lambda i: (0, 0)),
            pl.BlockSpec((HID, lat), lambda i: (0, 0)),
            pl.BlockSpec((1, lat), lambda i: (0, 0)),
        ],
        out_specs=[
            pl.BlockSpec((BR, lat), lambda i: (i, 0)),
            pl.BlockSpec((BR, lat), lambda i: (i, 0)),
        ],
        out_shape=[
            jax.ShapeDtypeStruct((N, lat), jnp.float32),
            jax.ShapeDtypeStruct((N, lat), jnp.float32),
        ],
    )(h, agg, w1, b1, w2, b2, wm, bm, ws, bs)


def kernel(x, edge_index, W1_0, b1_0, W2_0, b2_0, W1s, b1s, W2s, b2s, Wm, bm, Ws, bs):
    src = jnp.asarray(edge_index[0], jnp.int32)
    dst = jnp.asarray(edge_index[1], jnp.int32)
    perm = jnp.argsort(dst, stable=True)
    zpad = jnp.zeros((PAD,), jnp.int32)
    ssrc = jnp.concatenate([src[perm], zpad])
    sdst = jnp.concatenate([dst[perm], zpad])

    agg = _sc_aggregate128(x, ssrc, sdst)
    h = _tc_mlp0(x, agg, W1_0, b1_0.reshape(1, HID), W2_0, b2_0.reshape(1, HID))

    n_rest = W1s.shape[0]
    for i in range(n_rest - 1):
        agg = _sc_aggregate16(h, ssrc, sdst)
        h = _tc_mlp(h, agg, W1s[i], b1s[i].reshape(1, HID), W2s[i],
                    b2s[i].reshape(1, HID), relu=True)

    i = n_rest - 1
    agg = _sc_aggregate16(h, ssrc, sdst)
    lat = Wm.shape[1]
    mean, std = _tc_mlp_head(h, agg, W1s[i], b1s[i].reshape(1, HID), W2s[i],
                             b2s[i].reshape(1, HID), Wm, bm.reshape(1, lat),
                             Ws, bs.reshape(1, lat))
    return (mean, std)
